# baseline (speedup 1.0000x reference)
import jax
import jax.numpy as jnp
from jax import lax
from jax.experimental import pallas as pl
from jax.experimental.pallas import tpu as pltpu
from jax.experimental.pallas import tpu_sc as plsc

BATCH = 16384
DIM = 16
_NC = 2
_NS = 16
_NW = _NC * _NS
_BPW = BATCH // _NW


def _sc_body(idx_hbm, mean_hbm, logsd_hbm, z_hbm, out_hbm,
             idx_v, mean_v, sd_v, z_v, sem_m, sem_s):
    wid = lax.axis_index("s") * _NC + lax.axis_index("c")
    base = wid * _BPW
    pltpu.sync_copy(idx_hbm.at[pl.ds(base, _BPW)], idx_v)
    cm = pltpu.async_copy(mean_hbm.at[idx_v], mean_v, sem_m)
    cs = pltpu.async_copy(logsd_hbm.at[idx_v], sd_v, sem_s)
    pltpu.sync_copy(z_hbm.at[pl.ds(base, _BPW)], z_v)
    cm.wait()
    cs.wait()

    def body(i, carry):
        z_v[i, :] = mean_v[i, :] + z_v[i, :] * jnp.exp(sd_v[i, :])
        return carry
    lax.fori_loop(0, _BPW, body, 0)

    pltpu.sync_copy(z_v, out_hbm.at[pl.ds(base, _BPW)])


def kernel(indices, mean_table, log_sd_table, z, n_sample):
    b, s, d = z.shape
    z2 = z.reshape(b * s, d)
    idx = indices.astype(jnp.int32)

    mesh = plsc.VectorSubcoreMesh(core_axis_name="c", subcore_axis_name="s")
    out = pl.kernel(
        _sc_body,
        out_type=jax.ShapeDtypeStruct((BATCH, DIM), jnp.float32),
        mesh=mesh,
        compiler_params=pltpu.CompilerParams(use_tc_tiling_on_sc=False),
        scratch_types=[
            pltpu.VMEM((_BPW,), jnp.int32),
            pltpu.VMEM((_BPW, DIM), jnp.float32),
            pltpu.VMEM((_BPW, DIM), jnp.float32),
            pltpu.VMEM((_BPW, DIM), jnp.float32),
            pltpu.SemaphoreType.DMA,
            pltpu.SemaphoreType.DMA,
        ],
    )(idx, mean_table, log_sd_table, z2)
    return out.reshape(b, s, d)
